# baseline (speedup 1.0000x reference)
import functools

import jax
import jax.numpy as jnp
from jax import lax
from jax.experimental import pallas as pl
from jax.experimental.pallas import tpu as pltpu
from jax.experimental.pallas import tpu_sc as plsc

B = 16384
D = 64
V = 1000000
BLKV = 4096
NB4 = 61
NQ = BLKV * NB4
T64 = 4 * NQ
NTAIL = V - T64


def _words(x_ref):
    x = x_ref[...]
    hb = lax.bitcast_convert_type(x[:32, :].astype(jnp.bfloat16),
                                  jnp.uint16).astype(jnp.uint32)
    lb = lax.bitcast_convert_type(x[32:, :].astype(jnp.bfloat16),
                                  jnp.uint16).astype(jnp.uint32)
    return lax.bitcast_convert_type((hb << 16) | lb, jnp.float32)


def _detile_body(q0_ref, q1_ref, q2_ref, q3_ref, o_ref):
    w = jnp.concatenate([_words(q0_ref), _words(q1_ref),
                         _words(q2_ref), _words(q3_ref)], axis=0)
    o_ref[...] = w.T


def _detile(table_t):
    return pl.pallas_call(
        _detile_body,
        grid=(NB4,),
        in_specs=[
            pl.BlockSpec((D, BLKV), lambda i: (0, i)),
            pl.BlockSpec((D, BLKV), lambda i: (0, i + NB4)),
            pl.BlockSpec((D, BLKV), lambda i: (0, i + 2 * NB4)),
            pl.BlockSpec((D, BLKV), lambda i: (0, i + 3 * NB4)),
        ],
        out_specs=pl.BlockSpec((BLKV, 128), lambda i: (i, 0)),
        out_shape=jax.ShapeDtypeStruct((NQ, 128), jnp.float32),
        compiler_params=pltpu.CompilerParams(
            dimension_semantics=("arbitrary",)),
    )(table_t, table_t, table_t, table_t)


@functools.cache
def _make_sc_gather():
    info = plsc.get_sparse_core_info()
    NC, NS, L = info.num_cores, info.num_subcores, info.num_lanes
    NW = NC * NS
    bw = B // NW
    mesh = plsc.VectorSubcoreMesh(core_axis_name="c", subcore_axis_name="s")
    NCH = bw // 128

    @functools.partial(
        pl.kernel,
        mesh=mesh,
        out_type=jax.ShapeDtypeStruct((B, 128), jnp.float32),
        scratch_types=[
            pltpu.VMEM((bw,), jnp.int32),
            pltpu.VMEM((bw,), jnp.int32),
            pltpu.VMEM((bw, 128), jnp.float32),
            pltpu.SemaphoreType.DMA,
        ],
        compiler_params=pltpu.CompilerParams(use_tc_tiling_on_sc=True),
    )
    def gather_kernel(packed, idx_hbm, out_hbm, idx_v, q_v, rows_v, sem):
        wid = lax.axis_index("s") * NC + lax.axis_index("c")
        base = wid * bw
        pltpu.sync_copy(idx_hbm.at[pl.ds(base, bw)], idx_v)

        def qh_body(c, _):
            ids = idx_v[pl.ds(c * L, L)]
            g1 = jnp.where(ids >= NQ, 1, 0).astype(jnp.int32)
            g2 = jnp.where(ids >= 2 * NQ, 1, 0).astype(jnp.int32)
            g3 = jnp.where(ids >= 3 * NQ, 1, 0).astype(jnp.int32)
            q = ids - (g1 + g2 + g3) * NQ
            q_v[pl.ds(c * L, L)] = jnp.minimum(q, NQ - 1)
            return ()

        lax.fori_loop(0, bw // L, qh_body, (), unroll=True)
        pltpu.async_copy(packed.at[q_v], rows_v, sem).wait()
        pltpu.sync_copy(rows_v, out_hbm.at[pl.ds(base, bw)])

    return gather_kernel


def _mlp_body(wide_ref, code_ref, tail_ref, f_ref,
              W1_ref, b1_ref, W2_ref, b2_ref, W3a_ref, W3b_ref, b3_ref,
              g_ref, be_ref, W4_ref, b4_ref, o_ref):
    w = wide_ref[...]
    cu = code_ref[...]
    tm = cu >= 3.5
    c2 = cu - jnp.where(tm, 4.0, 0.0)
    pm = c2 >= 1.5
    c3 = c2 - jnp.where(pm, 2.0, 0.0)
    hm = c3 >= 0.5
    a64 = jnp.where(pm, w[:, 64:], w[:, :64])
    a32 = jnp.where(hm, a64[:, 32:], a64[:, :32])
    u = lax.bitcast_convert_type(a32, jnp.uint32)
    e_hi = lax.bitcast_convert_type(u & jnp.uint32(0xFFFF0000),
                                    jnp.float32)
    e_lo = lax.bitcast_convert_type(u << 16, jnp.float32)
    id_emb = jnp.concatenate([e_hi, e_lo], axis=1)
    id_emb = jnp.where(tm, tail_ref[...], id_emb)
    f = f_ref[...]
    h = jnp.maximum(
        jnp.dot(f, W1_ref[...], preferred_element_type=jnp.float32)
        + b1_ref[...], 0.0)
    fe = jnp.dot(h, W2_ref[...], preferred_element_type=jnp.float32) \
        + b2_ref[...]
    x = (jnp.dot(id_emb, W3a_ref[...], preferred_element_type=jnp.float32)
         + jnp.dot(fe, W3b_ref[...], preferred_element_type=jnp.float32)
         + b3_ref[...])
    mu = jnp.mean(x, axis=-1, keepdims=True)
    xc = x - mu
    var = jnp.mean(xc * xc, axis=-1, keepdims=True)
    x = g_ref[...] * xc * lax.rsqrt(var + 1e-5) + be_ref[...]
    x = jnp.maximum(x, 0.0)
    out = jnp.dot(x, W4_ref[...], preferred_element_type=jnp.float32) \
        + b4_ref[...]
    n2 = jnp.sum(out * out, axis=-1, keepdims=True)
    inv = 1.0 / jnp.maximum(jnp.sqrt(n2), 1e-12)
    o_ref[...] = out * inv


def _tc_mlp(wide, code, tail_emb, feats, W1, b1, W2, b2,
            W3a, W3b, b3, gamma, beta, W4, b4):
    BLK = 2048
    grid = (B // BLK,)
    F = feats.shape[1]

    row_spec = lambda w: pl.BlockSpec((BLK, w), lambda i: (i, 0))
    full = lambda a: pl.BlockSpec(a.shape, lambda i: (0,) * a.ndim)

    return pl.pallas_call(
        _mlp_body,
        grid=grid,
        in_specs=[
            row_spec(128), row_spec(1),
            row_spec(D), row_spec(F),
            full(W1), full(b1), full(W2), full(b2),
            full(W3a), full(W3b), full(b3),
            full(gamma), full(beta), full(W4), full(b4),
        ],
        out_specs=row_spec(D),
        out_shape=jax.ShapeDtypeStruct((B, D), jnp.float32),
        compiler_params=pltpu.CompilerParams(
            dimension_semantics=("arbitrary",)),
    )(wide, code, tail_emb, feats, W1, b1, W2, b2, W3a, W3b,
      b3, gamma, beta, W4, b4)


def kernel(user_ids, user_features, emb_table, W1, b1, W2, b2, W3, b3,
           gamma, beta, W4, b4):
    packed = _detile(emb_table.T)
    wide = _make_sc_gather()(packed, user_ids)
    g1 = (user_ids >= NQ).astype(jnp.int32)
    g2 = (user_ids >= 2 * NQ).astype(jnp.int32)
    g3 = (user_ids >= 3 * NQ).astype(jnp.int32)
    gt = (user_ids >= T64).astype(jnp.int32)
    hsel = g1 - g2 + g3
    code = (hsel + 2 * g2 + 4 * gt).astype(jnp.float32).reshape(B, 1)
    toff = user_ids - T64
    oh = (toff[:, None] == jnp.arange(NTAIL)[None, :]).astype(jnp.bfloat16)
    tail_emb = jnp.dot(oh, emb_table[T64:].astype(jnp.bfloat16),
                       preferred_element_type=jnp.float32)
    b1r = b1.reshape(1, -1)
    b2r = b2.reshape(1, -1)
    b3r = b3.reshape(1, -1)
    b4r = b4.reshape(1, -1)
    gr = gamma.reshape(1, -1)
    br = beta.reshape(1, -1)
    W3a = W3[:D]
    W3b = W3[D:]
    return _tc_mlp(wide, code, tail_emb, user_features, W1,
                   b1r, W2, b2r, W3a, W3b, b3r, gr, br, W4, b4r)


# speedup vs baseline: 1.0091x; 1.0091x over previous
import functools

import jax
import jax.numpy as jnp
from jax import lax
from jax.experimental import pallas as pl
from jax.experimental.pallas import tpu as pltpu
from jax.experimental.pallas import tpu_sc as plsc

B = 16384
D = 64
V = 1000000
BLKV = 4096
NB4 = 61
NQ = BLKV * NB4
T64 = 4 * NQ
NTAIL = V - T64


def _words(x_ref):
    x = x_ref[...]
    hb = lax.bitcast_convert_type(x[:32, :].astype(jnp.bfloat16),
                                  jnp.uint16).astype(jnp.uint32)
    lb = lax.bitcast_convert_type(x[32:, :].astype(jnp.bfloat16),
                                  jnp.uint16).astype(jnp.uint32)
    return lax.bitcast_convert_type((hb << 16) | lb, jnp.float32)


def _detile_body(q0_ref, q1_ref, q2_ref, q3_ref, o_ref):
    w = jnp.concatenate([_words(q0_ref), _words(q1_ref),
                         _words(q2_ref), _words(q3_ref)], axis=0)
    o_ref[...] = w.T


def _detile(table_t):
    return pl.pallas_call(
        _detile_body,
        grid=(NB4,),
        in_specs=[
            pl.BlockSpec((D, BLKV), lambda i: (0, i)),
            pl.BlockSpec((D, BLKV), lambda i: (0, i + NB4)),
            pl.BlockSpec((D, BLKV), lambda i: (0, i + 2 * NB4)),
            pl.BlockSpec((D, BLKV), lambda i: (0, i + 3 * NB4)),
        ],
        out_specs=pl.BlockSpec((BLKV, 128), lambda i: (i, 0)),
        out_shape=jax.ShapeDtypeStruct((NQ, 128), jnp.float32),
        compiler_params=pltpu.CompilerParams(
            dimension_semantics=("arbitrary",)),
    )(table_t, table_t, table_t, table_t)


@functools.cache
def _make_sc_gather():
    info = plsc.get_sparse_core_info()
    NC, NS, L = info.num_cores, info.num_subcores, info.num_lanes
    NW = NC * NS
    bw = B // NW
    mesh = plsc.VectorSubcoreMesh(core_axis_name="c", subcore_axis_name="s")
    NCH = bw // 128

    @functools.partial(
        pl.kernel,
        mesh=mesh,
        out_type=jax.ShapeDtypeStruct((B, 128), jnp.float32),
        scratch_types=[
            pltpu.VMEM((bw,), jnp.int32),
            pltpu.VMEM((NCH, 128), jnp.int32),
            pltpu.VMEM((bw, 128), jnp.float32),
            pltpu.SemaphoreType.DMA,
        ],
        compiler_params=pltpu.CompilerParams(use_tc_tiling_on_sc=True),
    )
    def gather_kernel(packed, idx_hbm, out_hbm, idx_v, q_v, rows_v, sem):
        wid = lax.axis_index("s") * NC + lax.axis_index("c")
        base = wid * bw
        pltpu.sync_copy(idx_hbm.at[pl.ds(base, bw)], idx_v)

        def qh_body(c, _):
            ids = idx_v[pl.ds(c * L, L)]
            g1 = jnp.where(ids >= NQ, 1, 0).astype(jnp.int32)
            g2 = jnp.where(ids >= 2 * NQ, 1, 0).astype(jnp.int32)
            g3 = jnp.where(ids >= 3 * NQ, 1, 0).astype(jnp.int32)
            q = ids - (g1 + g2 + g3) * NQ
            q_v[c // 8, pl.ds((c % 8) * L, L)] = jnp.minimum(q, NQ - 1)
            return ()

        lax.fori_loop(0, bw // L, qh_body, (), unroll=True)
        copies = [
            pltpu.async_copy(packed.at[q_v.at[k]],
                             rows_v.at[pl.ds(k * 128, 128)], sem)
            for k in range(NCH)
        ]
        for cp in copies:
            cp.wait()
        pltpu.sync_copy(rows_v, out_hbm.at[pl.ds(base, bw)])

    return gather_kernel


def _mlp_body(wide_ref, code_ref, tail_ref, f_ref,
              W1_ref, b1_ref, W2_ref, b2_ref, W3a_ref, W3b_ref, b3_ref,
              g_ref, be_ref, W4_ref, b4_ref, o_ref):
    w = wide_ref[...]
    cu = code_ref[...]
    tm = cu >= 3.5
    c2 = cu - jnp.where(tm, 4.0, 0.0)
    pm = c2 >= 1.5
    c3 = c2 - jnp.where(pm, 2.0, 0.0)
    hm = c3 >= 0.5
    a64 = jnp.where(pm, w[:, 64:], w[:, :64])
    a32 = jnp.where(hm, a64[:, 32:], a64[:, :32])
    u = lax.bitcast_convert_type(a32, jnp.uint32)
    e_hi = lax.bitcast_convert_type(u & jnp.uint32(0xFFFF0000),
                                    jnp.float32)
    e_lo = lax.bitcast_convert_type(u << 16, jnp.float32)
    id_emb = jnp.concatenate([e_hi, e_lo], axis=1)
    id_emb = jnp.where(tm, tail_ref[...], id_emb)
    f = f_ref[...]
    h = jnp.maximum(
        jnp.dot(f, W1_ref[...], preferred_element_type=jnp.float32)
        + b1_ref[...], 0.0)
    fe = jnp.dot(h, W2_ref[...], preferred_element_type=jnp.float32) \
        + b2_ref[...]
    x = (jnp.dot(id_emb, W3a_ref[...], preferred_element_type=jnp.float32)
         + jnp.dot(fe, W3b_ref[...], preferred_element_type=jnp.float32)
         + b3_ref[...])
    mu = jnp.mean(x, axis=-1, keepdims=True)
    xc = x - mu
    var = jnp.mean(xc * xc, axis=-1, keepdims=True)
    x = g_ref[...] * xc * lax.rsqrt(var + 1e-5) + be_ref[...]
    x = jnp.maximum(x, 0.0)
    out = jnp.dot(x, W4_ref[...], preferred_element_type=jnp.float32) \
        + b4_ref[...]
    n2 = jnp.sum(out * out, axis=-1, keepdims=True)
    inv = 1.0 / jnp.maximum(jnp.sqrt(n2), 1e-12)
    o_ref[...] = out * inv


def _tc_mlp(wide, code, tail_emb, feats, W1, b1, W2, b2,
            W3a, W3b, b3, gamma, beta, W4, b4):
    BLK = 2048
    grid = (B // BLK,)
    F = feats.shape[1]

    row_spec = lambda w: pl.BlockSpec((BLK, w), lambda i: (i, 0))
    full = lambda a: pl.BlockSpec(a.shape, lambda i: (0,) * a.ndim)

    return pl.pallas_call(
        _mlp_body,
        grid=grid,
        in_specs=[
            row_spec(128), row_spec(1),
            row_spec(D), row_spec(F),
            full(W1), full(b1), full(W2), full(b2),
            full(W3a), full(W3b), full(b3),
            full(gamma), full(beta), full(W4), full(b4),
        ],
        out_specs=row_spec(D),
        out_shape=jax.ShapeDtypeStruct((B, D), jnp.float32),
        compiler_params=pltpu.CompilerParams(
            dimension_semantics=("arbitrary",)),
    )(wide, code, tail_emb, feats, W1, b1, W2, b2, W3a, W3b,
      b3, gamma, beta, W4, b4)


def kernel(user_ids, user_features, emb_table, W1, b1, W2, b2, W3, b3,
           gamma, beta, W4, b4):
    packed = _detile(emb_table.T)
    wide = _make_sc_gather()(packed, user_ids)
    g1 = (user_ids >= NQ).astype(jnp.int32)
    g2 = (user_ids >= 2 * NQ).astype(jnp.int32)
    g3 = (user_ids >= 3 * NQ).astype(jnp.int32)
    gt = (user_ids >= T64).astype(jnp.int32)
    hsel = g1 - g2 + g3
    code = (hsel + 2 * g2 + 4 * gt).astype(jnp.float32).reshape(B, 1)
    toff = user_ids - T64
    oh = (toff[:, None] == jnp.arange(NTAIL)[None, :]).astype(jnp.bfloat16)
    tail_emb = jnp.dot(oh, emb_table[T64:].astype(jnp.bfloat16),
                       preferred_element_type=jnp.float32)
    b1r = b1.reshape(1, -1)
    b2r = b2.reshape(1, -1)
    b3r = b3.reshape(1, -1)
    b4r = b4.reshape(1, -1)
    gr = gamma.reshape(1, -1)
    br = beta.reshape(1, -1)
    W3a = W3[:D]
    W3b = W3[D:]
    return _tc_mlp(wide, code, tail_emb, user_features, W1,
                   b1r, W2, b2r, W3a, W3b, b3r, gr, br, W4, b4r)
